# baseline (speedup 1.0000x reference)
import functools

import jax
import jax.numpy as jnp
from jax import lax
from jax.experimental import pallas as pl
from jax.experimental.pallas import tpu as pltpu
from jax.experimental.pallas import tpu_sc as plsc

BATCH = 16384
EMBED_DIM = 32
LANES = 16

_info = plsc.get_sparse_core_info()
_NC, _NS = _info.num_cores, _info.num_subcores
NUM_WORKERS = _NC * _NS
BPW = BATCH // NUM_WORKERS


def _sc_kernel(users_hbm, items_hbm, ut_hbm, it_hbm, out_hbm,
               uidx_v, iidx_v, urows_v, irows_v, scores_v, sem_u, sem_i):
    wid = lax.axis_index("s") * _NC + lax.axis_index("c")
    base = wid * BPW

    pltpu.sync_copy(users_hbm.at[pl.ds(base, BPW)], uidx_v)
    pltpu.sync_copy(items_hbm.at[pl.ds(base, BPW)], iidx_v)

    cp_u = pltpu.async_copy(ut_hbm.at[uidx_v], urows_v, sem_u)
    cp_i = pltpu.async_copy(it_hbm.at[iidx_v], irows_v, sem_i)
    cp_u.wait()
    cp_i.wait()

    lane = lax.iota(jnp.int32, LANES)

    def body(b, carry):
        row = b * LANES + lane
        acc = jnp.zeros((LANES,), jnp.float32)
        for d in range(EMBED_DIM):
            col = jnp.full((LANES,), d, jnp.int32)
            u = plsc.load_gather(urows_v, [row, col])
            v = plsc.load_gather(irows_v, [row, col])
            acc = acc + u * v
        off = pl.multiple_of(b * LANES, LANES)
        scores_v[pl.ds(off, LANES)] = acc
        return carry

    lax.fori_loop(0, BPW // LANES, body, 0)

    pltpu.sync_copy(scores_v, out_hbm.at[pl.ds(base, BPW)])


@jax.jit
def _cml_scores(users, items, user_table, item_table):
    call = functools.partial(
        pl.kernel,
        mesh=plsc.VectorSubcoreMesh(core_axis_name="c", subcore_axis_name="s"),
        out_type=jax.ShapeDtypeStruct((BATCH,), jnp.float32),
        scratch_types=[
            pltpu.VMEM((BPW,), jnp.int32),
            pltpu.VMEM((BPW,), jnp.int32),
            pltpu.VMEM((BPW, EMBED_DIM), jnp.float32),
            pltpu.VMEM((BPW, EMBED_DIM), jnp.float32),
            pltpu.VMEM((BPW,), jnp.float32),
            pltpu.SemaphoreType.DMA,
            pltpu.SemaphoreType.DMA,
        ],
        compiler_params=pltpu.CompilerParams(
            needs_layout_passes=False, use_tc_tiling_on_sc=False),
    )(_sc_kernel)
    return call(users, items, user_table, item_table)


def kernel(users, items, user_table, item_table):
    users = users.astype(jnp.int32)
    items = items.astype(jnp.int32)
    return _cml_scores(users, items, user_table, item_table)


# speedup vs baseline: 3.9102x; 3.9102x over previous
import functools

import jax
import jax.numpy as jnp
from jax import lax
from jax.experimental import pallas as pl
from jax.experimental.pallas import tpu as pltpu
from jax.experimental.pallas import tpu_sc as plsc

BATCH = 16384
EMBED_DIM = 32
LANES = 16
RING = 4

_info = plsc.get_sparse_core_info()
_NC, _NS = _info.num_cores, _info.num_subcores
NUM_WORKERS = _NC * _NS
BPW = BATCH // NUM_WORKERS
CHUNKS = BPW // RING


def _extract_lane(vec, lane_sel, lane_iota):
    msk = lane_iota == jnp.broadcast_to(lane_sel, (LANES,))
    return jnp.sum(jnp.where(msk, vec, 0))


def _sc_kernel(users_hbm, items_hbm, ut_hbm, it_hbm, out_hbm,
               uidx_v, iidx_v, ubuf_v, ibuf_v, scores_v,
               usem, isem):
    wid = lax.axis_index("s") * _NC + lax.axis_index("c")
    base = wid * BPW
    lane = lax.iota(jnp.int32, LANES)
    d_lo = lane
    d_hi = lane + 16

    pltpu.sync_copy(users_hbm.at[pl.ds(base, BPW)], uidx_v)
    pltpu.sync_copy(items_hbm.at[pl.ds(base, BPW)], iidx_v)

    def entry_scalars(j16_off, j_lane):
        uvec = uidx_v[pl.ds(j16_off, LANES)]
        ivec = iidx_v[pl.ds(j16_off, LANES)]
        u = _extract_lane(uvec, j_lane, lane)
        i = _extract_lane(ivec, j_lane, lane)
        return u, i

    def fire(k, g):
        off16 = pl.multiple_of((g >> 2) * LANES, LANES)
        u, i = entry_scalars(off16, RING * (g & 3) + k)
        uct = pl.multiple_of((u >> 7) * 128, 128)
        ict = pl.multiple_of((i >> 7) * 128, 128)
        pltpu.async_copy(ut_hbm.at[:, pl.ds(uct, 128)], ubuf_v.at[k], usem[k])
        pltpu.async_copy(it_hbm.at[:, pl.ds(ict, 128)], ibuf_v.at[k], isem[k])

    for k in range(RING):
        fire(k, jnp.int32(0))

    def chunk(g, carry):
        off16 = pl.multiple_of((g >> 2) * LANES, LANES)
        acc = jnp.zeros((LANES,), jnp.float32)
        for k in range(RING):
            j_lane = RING * (g & 3) + k
            u, i = entry_scalars(off16, j_lane)
            lu = jnp.broadcast_to(u & 127, (LANES,))
            li = jnp.broadcast_to(i & 127, (LANES,))
            kk = jnp.broadcast_to(jnp.int32(k), (LANES,))
            pltpu.make_async_copy(
                ut_hbm.at[:, pl.ds(0, 128)], ubuf_v.at[k], usem[k]).wait()
            pltpu.make_async_copy(
                it_hbm.at[:, pl.ds(0, 128)], ibuf_v.at[k], isem[k]).wait()
            u0 = plsc.load_gather(ubuf_v, [kk, d_lo, lu])
            u1 = plsc.load_gather(ubuf_v, [kk, d_hi, lu])
            i0 = plsc.load_gather(ibuf_v, [kk, d_lo, li])
            i1 = plsc.load_gather(ibuf_v, [kk, d_hi, li])
            s = jnp.sum(u0 * i0 + u1 * i1)
            msk = lane == jnp.broadcast_to(j_lane, (LANES,))
            acc = acc + jnp.where(msk, jnp.broadcast_to(s, (LANES,)), 0.0)
            gn = jnp.minimum(g + 1, CHUNKS - 1)
            fire(k, gn)
        scores_v[pl.ds(off16, LANES)] = scores_v[pl.ds(off16, LANES)] + acc
        return carry

    def zero(b, carry):
        off = pl.multiple_of(b * LANES, LANES)
        scores_v[pl.ds(off, LANES)] = jnp.zeros((LANES,), jnp.float32)
        return carry

    lax.fori_loop(0, BPW // LANES, zero, 0)
    lax.fori_loop(0, CHUNKS, chunk, 0)

    for k in range(RING):
        pltpu.make_async_copy(
            ut_hbm.at[:, pl.ds(0, 128)], ubuf_v.at[k], usem[k]).wait()
        pltpu.make_async_copy(
            it_hbm.at[:, pl.ds(0, 128)], ibuf_v.at[k], isem[k]).wait()

    pltpu.sync_copy(scores_v, out_hbm.at[pl.ds(base, BPW)])


@jax.jit
def _cml_scores(users, items, ut_t, it_t):
    call = functools.partial(
        pl.kernel,
        mesh=plsc.VectorSubcoreMesh(core_axis_name="c", subcore_axis_name="s"),
        out_type=jax.ShapeDtypeStruct((BATCH,), jnp.float32),
        scratch_types=[
            pltpu.VMEM((BPW,), jnp.int32),
            pltpu.VMEM((BPW,), jnp.int32),
            pltpu.VMEM((RING, EMBED_DIM, 128), jnp.float32),
            pltpu.VMEM((RING, EMBED_DIM, 128), jnp.float32),
            pltpu.VMEM((BPW,), jnp.float32),
            [pltpu.SemaphoreType.DMA] * RING,
            [pltpu.SemaphoreType.DMA] * RING,
        ],
        compiler_params=pltpu.CompilerParams(
            needs_layout_passes=False, use_tc_tiling_on_sc=True),
    )(_sc_kernel)
    return call(users, items, ut_t, it_t)


def kernel(users, items, user_table, item_table):
    users = users.astype(jnp.int32)
    items = items.astype(jnp.int32)
    return _cml_scores(users, items, user_table.T, item_table.T)


# speedup vs baseline: 3.9934x; 1.0213x over previous
import functools

import jax
import jax.numpy as jnp
from jax import lax
from jax.experimental import pallas as pl
from jax.experimental.pallas import tpu as pltpu
from jax.experimental.pallas import tpu_sc as plsc

BATCH = 16384
EMBED_DIM = 32
LANES = 16
RING = 8
GROUP = 16 // RING

_info = plsc.get_sparse_core_info()
_NC, _NS = _info.num_cores, _info.num_subcores
NUM_WORKERS = _NC * _NS
BPW = BATCH // NUM_WORKERS
CHUNKS = BPW // RING


def _extract_lane(vec, lane_sel, lane_iota):
    msk = lane_iota == jnp.broadcast_to(lane_sel, (LANES,))
    return jnp.sum(jnp.where(msk, vec, 0))


def _sc_kernel(users_hbm, items_hbm, ut_hbm, it_hbm, out_hbm,
               uidx_v, iidx_v, ubuf_v, ibuf_v, scores_v,
               usem, isem):
    wid = lax.axis_index("s") * _NC + lax.axis_index("c")
    base = wid * BPW
    lane = lax.iota(jnp.int32, LANES)
    d_lo = lane
    d_hi = lane + 16

    pltpu.sync_copy(users_hbm.at[pl.ds(base, BPW)], uidx_v)
    pltpu.sync_copy(items_hbm.at[pl.ds(base, BPW)], iidx_v)

    def entry_scalars(j16_off, j_lane):
        uvec = uidx_v[pl.ds(j16_off, LANES)]
        ivec = iidx_v[pl.ds(j16_off, LANES)]
        u = _extract_lane(uvec, j_lane, lane)
        i = _extract_lane(ivec, j_lane, lane)
        return u, i

    def fire(k, g):
        off16 = pl.multiple_of((g // GROUP) * LANES, LANES)
        u, i = entry_scalars(off16, RING * (g % GROUP) + k)
        uct = pl.multiple_of((u >> 7) * 128, 128)
        ict = pl.multiple_of((i >> 7) * 128, 128)
        pltpu.async_copy(ut_hbm.at[:, pl.ds(uct, 128)], ubuf_v.at[k], usem[k])
        pltpu.async_copy(it_hbm.at[:, pl.ds(ict, 128)], ibuf_v.at[k], isem[k])

    for k in range(RING):
        fire(k, jnp.int32(0))

    def chunk(g, carry):
        off16 = pl.multiple_of((g // GROUP) * LANES, LANES)
        acc = jnp.zeros((LANES,), jnp.float32)
        for k in range(RING):
            j_lane = RING * (g % GROUP) + k
            u, i = entry_scalars(off16, j_lane)
            lu = jnp.broadcast_to(u & 127, (LANES,))
            li = jnp.broadcast_to(i & 127, (LANES,))
            kk = jnp.broadcast_to(jnp.int32(k), (LANES,))
            pltpu.make_async_copy(
                ut_hbm.at[:, pl.ds(0, 128)], ubuf_v.at[k], usem[k]).wait()
            pltpu.make_async_copy(
                it_hbm.at[:, pl.ds(0, 128)], ibuf_v.at[k], isem[k]).wait()
            u0 = plsc.load_gather(ubuf_v, [kk, d_lo, lu])
            u1 = plsc.load_gather(ubuf_v, [kk, d_hi, lu])
            i0 = plsc.load_gather(ibuf_v, [kk, d_lo, li])
            i1 = plsc.load_gather(ibuf_v, [kk, d_hi, li])
            s = jnp.sum(u0 * i0 + u1 * i1)
            msk = lane == jnp.broadcast_to(j_lane, (LANES,))
            acc = acc + jnp.where(msk, jnp.broadcast_to(s, (LANES,)), 0.0)
            gn = jnp.minimum(g + 1, CHUNKS - 1)
            fire(k, gn)
        scores_v[pl.ds(off16, LANES)] = scores_v[pl.ds(off16, LANES)] + acc
        return carry

    def zero(b, carry):
        off = pl.multiple_of(b * LANES, LANES)
        scores_v[pl.ds(off, LANES)] = jnp.zeros((LANES,), jnp.float32)
        return carry

    lax.fori_loop(0, BPW // LANES, zero, 0)
    lax.fori_loop(0, CHUNKS, chunk, 0)

    for k in range(RING):
        pltpu.make_async_copy(
            ut_hbm.at[:, pl.ds(0, 128)], ubuf_v.at[k], usem[k]).wait()
        pltpu.make_async_copy(
            it_hbm.at[:, pl.ds(0, 128)], ibuf_v.at[k], isem[k]).wait()

    pltpu.sync_copy(scores_v, out_hbm.at[pl.ds(base, BPW)])


@jax.jit
def _cml_scores(users, items, ut_t, it_t):
    call = functools.partial(
        pl.kernel,
        mesh=plsc.VectorSubcoreMesh(core_axis_name="c", subcore_axis_name="s"),
        out_type=jax.ShapeDtypeStruct((BATCH,), jnp.float32),
        scratch_types=[
            pltpu.VMEM((BPW,), jnp.int32),
            pltpu.VMEM((BPW,), jnp.int32),
            pltpu.VMEM((RING, EMBED_DIM, 128), jnp.float32),
            pltpu.VMEM((RING, EMBED_DIM, 128), jnp.float32),
            pltpu.VMEM((BPW,), jnp.float32),
            [pltpu.SemaphoreType.DMA] * RING,
            [pltpu.SemaphoreType.DMA] * RING,
        ],
        compiler_params=pltpu.CompilerParams(
            needs_layout_passes=False, use_tc_tiling_on_sc=True),
    )(_sc_kernel)
    return call(users, items, ut_t, it_t)


def kernel(users, items, user_table, item_table):
    users = users.astype(jnp.int32)
    items = items.astype(jnp.int32)
    return _cml_scores(users, items, user_table.T, item_table.T)
